# baseline (speedup 1.0000x reference)
import functools

import jax
import jax.numpy as jnp
from jax import lax
from jax.experimental import pallas as pl
from jax.experimental.pallas import tpu as pltpu
from jax.experimental.pallas import tpu_sc as plsc

_NC = 2
_NS = 16
_NW = _NC * _NS
_CHUNK = 128


@functools.lru_cache(maxsize=None)
def _build(V, D, B):
    b_per_w = B // _NW
    n_chunks = b_per_w // _CHUNK
    mesh = plsc.VectorSubcoreMesh(core_axis_name="c", subcore_axis_name="s")

    @functools.partial(
        pl.kernel,
        mesh=mesh,
        compiler_params=pltpu.CompilerParams(use_tc_tiling_on_sc=False),
        out_type=jax.ShapeDtypeStruct((B, D), jnp.float32),
        scratch_types=[
            pltpu.VMEM((n_chunks, _CHUNK), jnp.int32),
            pltpu.VMEM((_CHUNK, D), jnp.float32),
            pltpu.SemaphoreType.DMA,
        ],
    )
    def k(table_hbm, idx_hbm, out_hbm, idx_v, rows_v, sem):
        wid = lax.axis_index("s") * _NC + lax.axis_index("c")
        base = wid * b_per_w
        pltpu.sync_copy(idx_hbm.at[wid], idx_v)

        def body(j, carry):
            pltpu.async_copy(table_hbm.at[idx_v.at[j]], rows_v, sem).wait()
            off = pl.multiple_of(base + j * _CHUNK, _CHUNK)
            pltpu.sync_copy(rows_v, out_hbm.at[pl.ds(off, _CHUNK)])
            return carry

        lax.fori_loop(0, n_chunks, body, 0)

    return k


def kernel(table, inputs):
    D = table.shape[1]
    B = inputs.size
    idx = inputs.reshape(-1).astype(jnp.int32)
    pad = (-B) % (_NW * _CHUNK)
    if pad:
        idx = jnp.pad(idx, (0, pad))
    Bp = B + pad
    idx = idx.reshape(_NW, Bp // _NW // _CHUNK, _CHUNK)
    out = _build(table.shape[0], D, Bp)(table, idx)
    if pad:
        out = out[:B]
    return out.reshape(inputs.shape + (D,))


# speedup vs baseline: 1.0435x; 1.0435x over previous
import functools

import jax
import jax.numpy as jnp
from jax import lax
from jax.experimental import pallas as pl
from jax.experimental.pallas import tpu as pltpu
from jax.experimental.pallas import tpu_sc as plsc

_NC = 2
_NS = 16
_NW = _NC * _NS
_CHUNK = 128
_NBUF = 5
_G = 2


@functools.lru_cache(maxsize=None)
def _build(V, D, B):
    b_per_w = B // _NW
    n_chunks = b_per_w // _CHUNK
    assert n_chunks % _NBUF == 0
    mesh = plsc.VectorSubcoreMesh(core_axis_name="c", subcore_axis_name="s")

    @functools.partial(
        pl.kernel,
        mesh=mesh,
        compiler_params=pltpu.CompilerParams(use_tc_tiling_on_sc=False),
        out_type=jax.ShapeDtypeStruct((B, D), jnp.float32),
        scratch_types=[
            pltpu.VMEM((n_chunks, _CHUNK), jnp.int32),
            pltpu.VMEM((_NBUF, _CHUNK, D), jnp.float32),
        ]
        + [pltpu.SemaphoreType.DMA] * (2 * _NBUF),
    )
    def k(table_hbm, idx_hbm, out_hbm, idx_v, rows_v, *sems):
        gsems, wsems = sems[:_NBUF], sems[_NBUF:]
        wid = lax.axis_index("s") * _NC + lax.axis_index("c")
        base = wid * b_per_w
        pltpu.sync_copy(idx_hbm.at[wid], idx_v)

        def gather_start(c, b):
            pltpu.async_copy(table_hbm.at[idx_v.at[c]], rows_v.at[b], gsems[b])

        def gather_wait(c, b):
            pltpu.make_async_copy(
                table_hbm.at[idx_v.at[c]], rows_v.at[b], gsems[b]).wait()

        def write_start(c, b):
            pltpu.async_copy(
                rows_v.at[b], out_hbm.at[pl.ds(base + c * _CHUNK, _CHUNK)],
                wsems[b])

        def write_wait(c, b):
            pltpu.make_async_copy(
                rows_v.at[b], out_hbm.at[pl.ds(base + c * _CHUNK, _CHUNK)],
                wsems[b]).wait()

        for c in range(_G):
            gather_start(c, c)

        def group(g, carry):
            for b in range(_NBUF):
                j = g * _NBUF + b
                gather_wait(j, b)
                bc = (b + _G) % _NBUF
                cw = j + _G - _NBUF

                @pl.when(cw >= 0)
                def _():
                    write_wait(cw, bc)

                cg = j + _G

                @pl.when(cg < n_chunks)
                def _():
                    gather_start(cg, bc)

                write_start(j, b)
            return carry

        lax.fori_loop(0, n_chunks // _NBUF, group, 0)

        for t in range(_NBUF - _G):
            c = n_chunks - (_NBUF - _G) + t
            write_wait(c, c % _NBUF)

    return k


def kernel(table, inputs):
    D = table.shape[1]
    B = inputs.size
    idx = inputs.reshape(-1).astype(jnp.int32)
    pad = (-B) % (_NW * _CHUNK * _NBUF)
    if pad:
        idx = jnp.pad(idx, (0, pad))
    Bp = B + pad
    idx = idx.reshape(_NW, Bp // _NW // _CHUNK, _CHUNK)
    out = _build(table.shape[0], D, Bp)(table, idx)
    if pad:
        out = out[:B]
    return out.reshape(inputs.shape + (D,))


# speedup vs baseline: 1.0468x; 1.0031x over previous
import functools

import jax
import jax.numpy as jnp
from jax import lax
from jax.experimental import pallas as pl
from jax.experimental.pallas import tpu as pltpu
from jax.experimental.pallas import tpu_sc as plsc

_NC = 2
_NS = 16
_NW = _NC * _NS
_CHUNK = 128
_NBUF = 10
_G = 6


@functools.lru_cache(maxsize=None)
def _build(V, D, B):
    b_per_w = B // _NW
    n_chunks = b_per_w // _CHUNK
    assert n_chunks % _NBUF == 0
    mesh = plsc.VectorSubcoreMesh(core_axis_name="c", subcore_axis_name="s")

    @functools.partial(
        pl.kernel,
        mesh=mesh,
        compiler_params=pltpu.CompilerParams(use_tc_tiling_on_sc=False),
        out_type=jax.ShapeDtypeStruct((B, D), jnp.float32),
        scratch_types=[
            pltpu.VMEM((n_chunks, _CHUNK), jnp.int32),
            pltpu.VMEM((_NBUF, _CHUNK, D), jnp.float32),
        ]
        + [pltpu.SemaphoreType.DMA] * (2 * _NBUF),
    )
    def k(table_hbm, idx_hbm, out_hbm, idx_v, rows_v, *sems):
        gsems, wsems = sems[:_NBUF], sems[_NBUF:]
        wid = lax.axis_index("s") * _NC + lax.axis_index("c")
        base = wid * b_per_w
        pltpu.sync_copy(idx_hbm.at[wid], idx_v)

        def gather_start(c, b):
            pltpu.async_copy(table_hbm.at[idx_v.at[c]], rows_v.at[b], gsems[b])

        def gather_wait(c, b):
            pltpu.make_async_copy(
                table_hbm.at[idx_v.at[c]], rows_v.at[b], gsems[b]).wait()

        def write_start(c, b):
            pltpu.async_copy(
                rows_v.at[b], out_hbm.at[pl.ds(base + c * _CHUNK, _CHUNK)],
                wsems[b])

        def write_wait(c, b):
            pltpu.make_async_copy(
                rows_v.at[b], out_hbm.at[pl.ds(base + c * _CHUNK, _CHUNK)],
                wsems[b]).wait()

        for c in range(_G):
            gather_start(c, c)

        def group(g, carry):
            for b in range(_NBUF):
                j = g * _NBUF + b
                gather_wait(j, b)
                bc = (b + _G) % _NBUF
                cw = j + _G - _NBUF

                @pl.when(cw >= 0)
                def _():
                    write_wait(cw, bc)

                cg = j + _G

                @pl.when(cg < n_chunks)
                def _():
                    gather_start(cg, bc)

                write_start(j, b)
            return carry

        lax.fori_loop(0, n_chunks // _NBUF, group, 0)

        for t in range(_NBUF - _G):
            c = n_chunks - (_NBUF - _G) + t
            write_wait(c, c % _NBUF)

    return k


def kernel(table, inputs):
    D = table.shape[1]
    B = inputs.size
    idx = inputs.reshape(-1).astype(jnp.int32)
    pad = (-B) % (_NW * _CHUNK * _NBUF)
    if pad:
        idx = jnp.pad(idx, (0, pad))
    Bp = B + pad
    idx = idx.reshape(_NW, Bp // _NW // _CHUNK, _CHUNK)
    out = _build(table.shape[0], D, Bp)(table, idx)
    if pad:
        out = out[:B]
    return out.reshape(inputs.shape + (D,))
